# baseline (speedup 1.0000x reference)
import jax
import jax.numpy as jnp
from jax import lax
from jax.experimental import pallas as pl
from jax.experimental.pallas import tpu as pltpu
from jax.experimental.pallas import tpu_sc as plsc

V, D, N, C = 1000000, 64, 16384, 20

NC, NS = 2, 16
NW = NC * NS
WPW = N // NW
W = 8
CW = W * C
NCH = WPW // W
K = 4
NREG = D // 16
ROW = 2 * D
TB = 32768
HB = TB // 2
TBL = TB.bit_length() - 1
NTB = -(-V // TB)
WIDE_R = NTB * HB


def _tr_body(x_ref, o_ref):
    o_ref[:, 0:D] = x_ref[:, 0:HB].T
    o_ref[:, D:ROW] = x_ref[:, HB:TB].T


@jax.jit
def _widen(out_table):
    t = out_table.T
    return pl.pallas_call(
        _tr_body,
        grid=(NTB,),
        in_specs=[pl.BlockSpec((D, TB), lambda i: (0, i))],
        out_specs=pl.BlockSpec((HB, ROW), lambda i: (i, 0)),
        out_shape=jax.ShapeDtypeStruct((WIDE_R, ROW), jnp.float32),
    )(t)


def _sc_body(temb_hbm, pcf_hbm, ncf_hbm, wide_hbm,
             sims_hbm,
             rbuf, hbuf, cbuf, tbuf, simbuf,
             sem_i0, sem_i1, sem_i2, sem_i3,
             sem_d0, sem_d1, sem_d2, sem_d3):
    wid = lax.axis_index("s") * NC + lax.axis_index("c")
    base = wid * WPW
    sem_i = (sem_i0, sem_i1, sem_i2, sem_i3)
    sem_d = (sem_d0, sem_d1, sem_d2, sem_d3)

    def run_phase(cr_hbm, ph):
        def issue_idx(g, b):
            co = (base + g * W) * C
            pltpu.async_copy(cr_hbm.at[pl.ds(co, CW)], rbuf.at[b], sem_i[b])

        def wait_idx(b):
            pltpu.make_async_copy(cr_hbm.at[pl.ds(0, CW)], rbuf.at[b],
                                  sem_i[b]).wait()

        def transform_idx(b):
            def tb_body(k, _):
                sl = pl.ds(k * 16, 16)
                v = rbuf[b, sl]
                rbuf[b, sl] = ((v >> TBL) << (TBL - 1)) + (v & (HB - 1))
                hbuf[b, sl] = ((v >> (TBL - 1)) & 1) * D
                return 0

            lax.fori_loop(0, CW // 16, tb_body, 0, unroll=False)

        def issue_gathers(g, b):
            wb = base + g * W
            pltpu.async_copy(wide_hbm.at[rbuf.at[b, pl.ds(0, 128)]],
                             cbuf.at[b, pl.ds(0, 128)], sem_d[b])
            pltpu.async_copy(wide_hbm.at[rbuf.at[b, pl.ds(128, 32)]],
                             cbuf.at[b, pl.ds(128, 32)], sem_d[b])
            pltpu.async_copy(temb_hbm.at[pl.ds(wb, W), :], tbuf.at[b],
                             sem_d[b])

        def wait_data(b):
            pltpu.make_async_copy(wide_hbm.at[pl.ds(0, CW)], cbuf.at[b],
                                  sem_d[b]).wait()
            pltpu.make_async_copy(temb_hbm.at[pl.ds(0, W), :], tbuf.at[b],
                                  sem_d[b]).wait()

        def compute(g, b, lane_base, sv0):
            lanes = lax.iota(jnp.int32, 16)

            def word_body(w, sv):
                r0 = w * C
                hv1 = hbuf[b, pl.ds(r0, 16)]
                hv2 = hbuf[b, pl.ds(r0 + 4, 16)]
                acc = [None] * NREG
                for c in range(C):
                    hc = hv1[c] if c < 16 else hv2[c - 4]
                    hc = pl.multiple_of(hc, 64)
                    for j in range(NREG):
                        x = cbuf[b, r0 + c, pl.ds(hc + j * 16, 16)]
                        acc[j] = x if acc[j] is None else acc[j] + x
                prod = [acc[j] * tbuf[b, w, pl.ds(j * 16, 16)]
                        for j in range(NREG)]
                q = (prod[0] + prod[1]) + (prod[2] + prod[3])
                for k in (1, 2, 4, 8):
                    q = q + jnp.take_along_axis(
                        q, jnp.bitwise_xor(lanes, k), axis=0)
                return jnp.where(lanes == w + lane_base, q, sv)

            return lax.fori_loop(0, W, word_body, sv0, unroll=False)

        for c0 in range(K):
            issue_idx(c0, c0)
        for c0 in range(3):
            wait_idx(c0)
            transform_idx(c0)
            issue_gathers(c0, c0)

        def chunk_quad(gg, sv):
            for sub in range(K):
                g = gg * K + sub
                b = sub
                nb = (sub + 3) % K

                @pl.when(g < NCH - 3)
                def _():
                    wait_idx(nb)
                    transform_idx(nb)
                    issue_gathers(g + 3, nb)

                wait_data(b)
                sv = compute(g, b, (sub % 2) * W, sv)
                if sub % 2 == 1:
                    simbuf[pl.ds(ph * WPW + (g - 1) * W, 16)] = sv
                    sv = jnp.zeros((16,), jnp.float32)

                @pl.when(g < NCH - K)
                def _():
                    issue_idx(g + K, b)

            return sv

        lax.fori_loop(0, NCH // K, chunk_quad, jnp.zeros((16,), jnp.float32),
                      unroll=False)

    run_phase(pcf_hbm, 0)
    run_phase(ncf_hbm, 1)

    pltpu.sync_copy(simbuf.at[pl.ds(0, WPW)], sims_hbm.at[pl.ds(base, WPW)])
    pltpu.sync_copy(simbuf.at[pl.ds(WPW, WPW)],
                    sims_hbm.at[pl.ds(N + base, WPW)])


@jax.jit
def _sc_sims(words, pos_contexts, neg_contexts, emb_table, out_table):
    wi = words.astype(jnp.int32)
    pcf = pos_contexts.astype(jnp.int32).reshape(N * C)
    ncf = neg_contexts.astype(jnp.int32).reshape(N * C)
    wide = _widen(out_table)
    temb = jnp.take(emb_table, wi, axis=0, mode="clip")

    mesh = plsc.VectorSubcoreMesh(core_axis_name="c", subcore_axis_name="s",
                                  num_cores=NC, num_subcores=NS)
    f = pl.kernel(
        _sc_body,
        out_type=jax.ShapeDtypeStruct((2 * N,), jnp.float32),
        mesh=mesh,
        compiler_params=pltpu.CompilerParams(use_tc_tiling_on_sc=False),
        scratch_types=[
            pltpu.VMEM((K, CW), jnp.int32),
            pltpu.VMEM((K, CW), jnp.int32),
            pltpu.VMEM((K, CW, ROW), jnp.float32),
            pltpu.VMEM((K, W, D), jnp.float32),
            pltpu.VMEM((2 * WPW,), jnp.float32),
            pltpu.SemaphoreType.DMA,
            pltpu.SemaphoreType.DMA,
            pltpu.SemaphoreType.DMA,
            pltpu.SemaphoreType.DMA,
            pltpu.SemaphoreType.DMA,
            pltpu.SemaphoreType.DMA,
            pltpu.SemaphoreType.DMA,
            pltpu.SemaphoreType.DMA,
        ],
    )
    return f(temb, pcf, ncf, wide)


def _loss_body(s_ref, o_ref):
    s = s_ref[...]
    ps = s[0:128, :]
    ns = s[128:256, :]
    pos_loss = jax.nn.log_sigmoid(ps)
    neg_loss = jax.nn.log_sigmoid(-ns)
    o_ref[0, 0] = -jnp.sum(pos_loss + neg_loss) / jnp.float32(N)


@jax.jit
def _tc_loss(sims):
    out = pl.pallas_call(
        _loss_body,
        out_shape=jax.ShapeDtypeStruct((1, 1), jnp.float32),
        out_specs=pl.BlockSpec(memory_space=pltpu.SMEM),
    )(sims.reshape(256, 128))
    return out[0, 0]


def kernel(words, pos_contexts, neg_contexts, emb_table, out_table):
    sims = _sc_sims(words, pos_contexts, neg_contexts, emb_table, out_table)
    return _tc_loss(sims)


# speedup vs baseline: 1.0358x; 1.0358x over previous
import jax
import jax.numpy as jnp
from jax import lax
from jax.experimental import pallas as pl
from jax.experimental.pallas import tpu as pltpu
from jax.experimental.pallas import tpu_sc as plsc

V, D, N, C = 1000000, 64, 16384, 20

NC, NS = 2, 16
NW = NC * NS
WPW = N // NW
W = 8
CW = W * C
NCH = WPW // W
K = 4
NREG = D // 16
ROW = 2 * D
TB = 32768
HB = TB // 2
TBL = TB.bit_length() - 1
NTB = -(-V // TB)
WIDE_R = NTB * HB


def _tr_body(x_ref, o_ref):
    o_ref[:, 0:D] = x_ref[:, 0:HB].T
    o_ref[:, D:ROW] = x_ref[:, HB:TB].T


@jax.jit
def _widen(out_table):
    t = out_table.T
    return pl.pallas_call(
        _tr_body,
        grid=(NTB,),
        in_specs=[pl.BlockSpec((D, TB), lambda i: (0, i))],
        out_specs=pl.BlockSpec((HB, ROW), lambda i: (i, 0)),
        out_shape=jax.ShapeDtypeStruct((WIDE_R, ROW), jnp.float32),
    )(t)


def _sc_body(temb_hbm, pcf_hbm, ncf_hbm, wide_hbm,
             sims_hbm,
             rbuf, hbuf, cbuf, tbuf, simbuf,
             sem_i0, sem_i1, sem_i2, sem_i3,
             sem_d0, sem_d1, sem_d2, sem_d3):
    wid = lax.axis_index("s") * NC + lax.axis_index("c")
    base = wid * WPW
    sem_i = (sem_i0, sem_i1, sem_i2, sem_i3)
    sem_d = (sem_d0, sem_d1, sem_d2, sem_d3)

    def run_phase(cr_hbm, ph):
        def issue_idx(g, b):
            co = (base + g * W) * C
            pltpu.async_copy(cr_hbm.at[pl.ds(co, CW)], rbuf.at[b], sem_i[b])

        def wait_idx(b):
            pltpu.make_async_copy(cr_hbm.at[pl.ds(0, CW)], rbuf.at[b],
                                  sem_i[b]).wait()

        def transform_idx(b):
            def tb_body(k, _):
                sl = pl.ds(k * 16, 16)
                v = rbuf[b, sl]
                rbuf[b, sl] = ((v >> TBL) << (TBL - 1)) + (v & (HB - 1))
                hbuf[b, sl] = ((v >> (TBL - 1)) & 1) * D
                return 0

            lax.fori_loop(0, CW // 16, tb_body, 0, unroll=False)

        def issue_gathers(g, b):
            wb = base + g * W
            pltpu.async_copy(wide_hbm.at[rbuf.at[b, pl.ds(0, 128)]],
                             cbuf.at[b, pl.ds(0, 128)], sem_d[b])
            pltpu.async_copy(wide_hbm.at[rbuf.at[b, pl.ds(128, 32)]],
                             cbuf.at[b, pl.ds(128, 32)], sem_d[b])
            pltpu.async_copy(temb_hbm.at[pl.ds(wb, W), :], tbuf.at[b],
                             sem_d[b])

        def wait_data(b):
            pltpu.make_async_copy(wide_hbm.at[pl.ds(0, CW)], cbuf.at[b],
                                  sem_d[b]).wait()
            pltpu.make_async_copy(temb_hbm.at[pl.ds(0, W), :], tbuf.at[b],
                                  sem_d[b]).wait()

        def compute(g, b, lane_base, sv0):
            lanes = lax.iota(jnp.int32, 16)

            def word_body(w, sv):
                r0 = w * C
                hv1 = hbuf[b, pl.ds(r0, 16)]
                hv2 = hbuf[b, pl.ds(r0 + 4, 16)]
                acc = [None] * NREG
                for c in range(C):
                    hc = hv1[c] if c < 16 else hv2[c - 4]
                    hc = pl.multiple_of(hc, 64)
                    for j in range(NREG):
                        x = cbuf[b, r0 + c, pl.ds(hc + j * 16, 16)]
                        acc[j] = x if acc[j] is None else acc[j] + x
                prod = [acc[j] * tbuf[b, w, pl.ds(j * 16, 16)]
                        for j in range(NREG)]
                q = (prod[0] + prod[1]) + (prod[2] + prod[3])
                for k in (1, 2, 4, 8):
                    q = q + jnp.take_along_axis(
                        q, jnp.bitwise_xor(lanes, k), axis=0)
                return jnp.where(lanes == w + lane_base, q, sv)

            return lax.fori_loop(0, W, word_body, sv0, unroll=False)

        for c0 in range(K):
            issue_idx(c0, c0)
        for c0 in range(2):
            wait_idx(c0)
            transform_idx(c0)
            issue_gathers(c0, c0)

        def chunk_quad(gg, sv):
            for sub in range(K):
                g = gg * K + sub
                b = sub
                nb = (sub + 2) % K

                @pl.when(g < NCH - 2)
                def _():
                    wait_idx(nb)
                    transform_idx(nb)
                    issue_gathers(g + 2, nb)

                wait_data(b)
                sv = compute(g, b, (sub % 2) * W, sv)
                if sub % 2 == 1:
                    simbuf[pl.ds(ph * WPW + (g - 1) * W, 16)] = sv
                    sv = jnp.zeros((16,), jnp.float32)

                @pl.when(g < NCH - K)
                def _():
                    issue_idx(g + K, b)

            return sv

        lax.fori_loop(0, NCH // K, chunk_quad, jnp.zeros((16,), jnp.float32),
                      unroll=False)

    run_phase(pcf_hbm, 0)
    run_phase(ncf_hbm, 1)

    pltpu.sync_copy(simbuf.at[pl.ds(0, WPW)], sims_hbm.at[pl.ds(base, WPW)])
    pltpu.sync_copy(simbuf.at[pl.ds(WPW, WPW)],
                    sims_hbm.at[pl.ds(N + base, WPW)])


@jax.jit
def _sc_sims(words, pos_contexts, neg_contexts, emb_table, out_table):
    wi = words.astype(jnp.int32)
    pcf = pos_contexts.astype(jnp.int32).reshape(N * C)
    ncf = neg_contexts.astype(jnp.int32).reshape(N * C)
    wide = _widen(out_table)
    temb = jnp.take(emb_table, wi, axis=0, mode="clip")

    mesh = plsc.VectorSubcoreMesh(core_axis_name="c", subcore_axis_name="s",
                                  num_cores=NC, num_subcores=NS)
    f = pl.kernel(
        _sc_body,
        out_type=jax.ShapeDtypeStruct((2 * N,), jnp.float32),
        mesh=mesh,
        compiler_params=pltpu.CompilerParams(use_tc_tiling_on_sc=False),
        scratch_types=[
            pltpu.VMEM((K, CW), jnp.int32),
            pltpu.VMEM((K, CW), jnp.int32),
            pltpu.VMEM((K, CW, ROW), jnp.float32),
            pltpu.VMEM((K, W, D), jnp.float32),
            pltpu.VMEM((2 * WPW,), jnp.float32),
            pltpu.SemaphoreType.DMA,
            pltpu.SemaphoreType.DMA,
            pltpu.SemaphoreType.DMA,
            pltpu.SemaphoreType.DMA,
            pltpu.SemaphoreType.DMA,
            pltpu.SemaphoreType.DMA,
            pltpu.SemaphoreType.DMA,
            pltpu.SemaphoreType.DMA,
        ],
    )
    return f(temb, pcf, ncf, wide)


def _loss_body(s_ref, o_ref):
    s = s_ref[...]
    ps = s[0:128, :]
    ns = s[128:256, :]
    pos_loss = jax.nn.log_sigmoid(ps)
    neg_loss = jax.nn.log_sigmoid(-ns)
    o_ref[0, 0] = -jnp.sum(pos_loss + neg_loss) / jnp.float32(N)


@jax.jit
def _tc_loss(sims):
    out = pl.pallas_call(
        _loss_body,
        out_shape=jax.ShapeDtypeStruct((1, 1), jnp.float32),
        out_specs=pl.BlockSpec(memory_space=pltpu.SMEM),
    )(sims.reshape(256, 128))
    return out[0, 0]


def kernel(words, pos_contexts, neg_contexts, emb_table, out_table):
    sims = _sc_sims(words, pos_contexts, neg_contexts, emb_table, out_table)
    return _tc_loss(sims)
